# baseline (speedup 1.0000x reference)
import functools

import jax
import jax.numpy as jnp
from jax import lax
from jax.experimental import pallas as pl
from jax.experimental.pallas import tpu as pltpu
from jax.experimental.pallas import tpu_sc as plsc

_WORDDIM = 128
_SMALLDIM = 32
_NLOOK = 6
_SMALLVOCAB = 1000
_INP = 6 * _WORDDIM + 6 * _SMALLDIM + 6 * _SMALLDIM
_HID = 200

_mesh = plsc.VectorSubcoreMesh(core_axis_name="c", subcore_axis_name="s")


@functools.partial(
    pl.kernel,
    mesh=_mesh,
    compiler_params=pltpu.CompilerParams(needs_layout_passes=False),
    out_type=jax.ShapeDtypeStruct((_INP // 128, 128), jnp.float32),
    scratch_types=[
        pltpu.VMEM((_NLOOK,), jnp.int32),
        pltpu.VMEM((16,), jnp.int32),
        pltpu.VMEM((16,), jnp.int32),
        pltpu.VMEM((_INP // 128, 128), jnp.float32),
        pltpu.VMEM((_SMALLVOCAB * _SMALLDIM // 128, 128), jnp.float32),
        pltpu.VMEM((_SMALLVOCAB * _SMALLDIM // 128, 128), jnp.float32),
        pltpu.SemaphoreType.DMA,
        pltpu.SemaphoreType.DMA,
        pltpu.SemaphoreType.DMA,
    ],
)
def _sc_gather(wordid, posid, labelid, wordembed, posembed, labelembed,
               out, widx, pidx, lidx, jfull, ptab, ltab, sw, sp, sl):
    c = lax.axis_index("c")
    s = lax.axis_index("s")

    @pl.when(jnp.logical_and(c == 0, s == 0))
    def _():
        pltpu.sync_copy(wordid, widx)
        pltpu.sync_copy(posid, pidx.at[pl.ds(0, _NLOOK)])
        pltpu.sync_copy(labelid, lidx.at[pl.ds(0, _NLOOK)])
        cw = pltpu.async_copy(wordembed.at[widx], jfull.at[pl.ds(0, 6)], sw)
        cp = pltpu.async_copy(posembed, ptab, sp)
        cl = pltpu.async_copy(labelembed, ltab, sl)
        cw.wait()
        cp.wait()
        cl.wait()
        iota = lax.iota(jnp.int32, 16)
        for t, (tab, idx) in enumerate(((ptab, pidx), (ltab, lidx))):
            idxvec = idx[...]
            for r in range(_NLOOK):
                row = idxvec[r]
                for h in range(2):
                    flat = row * _SMALLDIM + iota + 16 * h
                    vals = plsc.load_gather(
                        tab, [lax.shift_right_logical(flat, 7), flat & 127])
                    off = 6 * _WORDDIM + t * 6 * _SMALLDIM + r * _SMALLDIM + 16 * h
                    jfull[off // 128, pl.ds(off % 128, 16)] = vals
        pltpu.sync_copy(jfull, out)


def _mlp_body(x_ref, w1_ref, b1_ref, w2_ref, b2_ref, o_ref):
    h = jnp.dot(x_ref[...], w1_ref[...], preferred_element_type=jnp.float32,
                precision=jax.lax.Precision.HIGHEST)
    h = h + b1_ref[...]
    h3 = h * h * h
    o = jnp.dot(h3, w2_ref[...], preferred_element_type=jnp.float32,
                precision=jax.lax.Precision.HIGHEST)
    o_ref[...] = o + b2_ref[...]


_mlp = pl.pallas_call(
    _mlp_body,
    out_shape=jax.ShapeDtypeStruct((1, 3), jnp.float32),
)


def kernel(wordid, posid, labelid, wordembed, posembed, labelembed,
           W1, b1, W2, b2):
    joined = _sc_gather(wordid.astype(jnp.int32), posid.astype(jnp.int32),
                        labelid.astype(jnp.int32),
                        wordembed,
                        posembed.reshape(_SMALLVOCAB * _SMALLDIM // 128, 128),
                        labelembed.reshape(_SMALLVOCAB * _SMALLDIM // 128, 128))
    x = joined.reshape(1, _INP)
    return _mlp(x, W1, b1.reshape(1, _HID), W2, b2.reshape(1, 3))
